# baseline (speedup 1.0000x reference)
import functools

import jax
import jax.numpy as jnp
from jax import lax
from jax.experimental import pallas as pl
from jax.experimental.pallas import tpu as pltpu
from jax.experimental.pallas import tpu_sc as plsc

_K = 8192
_D = 256
_M = 4608
_G = 1
_MG = _M // _G
_BK = 4096
_NSTEPS = _K // _BK
_LANES = 128
_BIG = 2 ** 30

_NC = 2
_NS = 16
_NW = _NC * _NS
_ROWS_PER_W = _MG // _NW
_NCHUNK = 3
_IDX_CHUNK = _ROWS_PER_W // _NCHUNK


def _argmin_body(x_ref, ct_ref, tok_ref, minv_ref, mini_ref, xm2_ref,
                 x2_ref):
    k = pl.program_id(0)

    @pl.when(k == 0)
    def _init():
        minv_ref[...] = jnp.full(minv_ref.shape, jnp.inf, jnp.float32)
        mini_ref[...] = jnp.zeros(mini_ref.shape, jnp.float32)
        xm2_ref[...] = x_ref[...] * (-2.0)
        x2_ref[...] = jnp.sum(jnp.square(x_ref[...]), axis=1, keepdims=True)

    m = x_ref.shape[0]
    lane = lax.broadcasted_iota(jnp.int32, (m, _LANES), 1).astype(jnp.float32)
    base = (k * _BK).astype(jnp.float32)
    x2v = x2_ref[...]
    _H = 256

    def _tree(vals, idxs):
        while len(vals) > 1:
            nv, ni = [], []
            for a in range(0, len(vals), 2):
                s = vals[a + 1] < vals[a]
                nv.append(jnp.minimum(vals[a], vals[a + 1]))
                ni.append(jnp.where(s, idxs[a + 1], idxs[a]))
            vals, idxs = nv, ni
        return vals[0], idxs[0]

    def _chunk(h):
        cc = ct_ref[h * _H:(h + 1) * _H, :]
        c2b = jnp.transpose(
            jnp.sum(jnp.square(cc), axis=1, keepdims=True))
        xc = lax.dot_general(
            xm2_ref[...], cc, (((1,), (1,)), ((), ())),
            preferred_element_type=jnp.float32)
        vals = [(x2v + xc[:, t * _LANES:(t + 1) * _LANES])
                + c2b[:, t * _LANES:(t + 1) * _LANES]
                for t in range(_H // _LANES)]
        idxs = [lane + (h * _H + t * _LANES) for t in range(_H // _LANES)]
        return _tree(vals, idxs)

    parts = [_chunk(h) for h in range(_BK // _H)]
    bv, bi = _tree([p[0] for p in parts], [p[1] for p in parts])
    cur_v = minv_ref[...]
    u = bv < cur_v
    minv_ref[...] = jnp.minimum(bv, cur_v)
    mini_ref[...] = jnp.where(u, bi + base, mini_ref[...])

    @pl.when(k == _NSTEPS - 1)
    def _finish():
        mv = minv_ref[...]
        mn = jnp.min(mv, axis=1, keepdims=True)
        cand = jnp.where(mv == mn, mini_ref[...], jnp.float32(_BIG))
        tok = jnp.min(cand, axis=1, keepdims=True)
        tok_ref[...] = tok.astype(jnp.int32)


def _tokens(x, cb):
    m = x.shape[0]
    return pl.pallas_call(
        _argmin_body,
        grid=(_NSTEPS,),
        in_specs=[
            pl.BlockSpec((m, _D), lambda k: (0, 0)),
            pl.BlockSpec((_BK, _D), lambda k: (k, 0)),
        ],
        out_specs=pl.BlockSpec((m, 1), lambda k: (0, 0)),
        out_shape=jax.ShapeDtypeStruct((m, 1), jnp.int32),
        scratch_shapes=[
            pltpu.VMEM((m, _LANES), jnp.float32),
            pltpu.VMEM((m, _LANES), jnp.float32),
            pltpu.VMEM((m, _D), jnp.float32),
            pltpu.VMEM((m, 1), jnp.float32),
        ],
    )(x, cb)


def _sc_gather_body(table_hbm, idx_hbm, out_hbm, idx_v, rows_v, gsem, osem):
    wid = lax.axis_index("s") * _NC + lax.axis_index("c")
    pltpu.sync_copy(idx_hbm.at[wid], idx_v)
    gathers = [
        pltpu.async_copy(
            table_hbm.at[idx_v.at[c]],
            rows_v.at[pl.ds(c * _IDX_CHUNK, _IDX_CHUNK)], gsem)
        for c in range(_NCHUNK)
    ]
    outs = []
    base = _ROWS_PER_W * wid
    for c in range(_NCHUNK):
        gathers[c].wait()
        outs.append(pltpu.async_copy(
            rows_v.at[pl.ds(c * _IDX_CHUNK, _IDX_CHUNK)],
            out_hbm.at[pl.ds(base + c * _IDX_CHUNK, _IDX_CHUNK)], osem))
    for cp in outs:
        cp.wait()


@functools.cache
def _sc_gather():
    return pl.kernel(
        _sc_gather_body,
        out_type=jax.ShapeDtypeStruct((_MG, _D), jnp.float32),
        mesh=plsc.VectorSubcoreMesh(core_axis_name="c", subcore_axis_name="s"),
        scratch_types=[
            pltpu.VMEM((_NCHUNK, _IDX_CHUNK), jnp.int32),
            pltpu.VMEM((_ROWS_PER_W, _D), jnp.float32),
            pltpu.SemaphoreType.DMA,
            pltpu.SemaphoreType.DMA,
        ],
    )


def kernel(inputs, codebook, training):
    shape = inputs.shape
    x = inputs.reshape(_M, _D)
    embs = []
    for g in range(_G):
        xg = x[g * _MG:(g + 1) * _MG]
        tok = _tokens(xg, codebook)
        idx = tok.reshape(_NW, _NCHUNK, _IDX_CHUNK)
        embs.append(_sc_gather()(codebook, idx))
    del training
    return jnp.concatenate(embs, axis=0).reshape(shape)


# speedup vs baseline: 1.0091x; 1.0091x over previous
import functools

import jax
import jax.numpy as jnp
from jax import lax
from jax.experimental import pallas as pl
from jax.experimental.pallas import tpu as pltpu
from jax.experimental.pallas import tpu_sc as plsc

_K = 8192
_D = 256
_M = 4608
_MG = _M
_BK = 4096
_NSTEPS = _K // _BK
_LANES = 128
_BIG = 2 ** 30

_NC = 2
_NS = 16
_NW = _NC * _NS
_ROWS_PER_W = _MG // _NW
_NCHUNK = 2
_IDX_CHUNK = _ROWS_PER_W // _NCHUNK


def _argmin_body(x_ref, ct_ref, tok_ref, minv_ref, mini_ref, xm2_ref,
                 x2_ref):
    k = pl.program_id(0)

    @pl.when(k == 0)
    def _init():
        minv_ref[...] = jnp.full(minv_ref.shape, jnp.inf, jnp.float32)
        mini_ref[...] = jnp.zeros(mini_ref.shape, jnp.float32)
        xm2_ref[...] = x_ref[...] * (-2.0)
        x2_ref[...] = jnp.sum(jnp.square(x_ref[...]), axis=1, keepdims=True)

    m = x_ref.shape[0]
    lane = lax.broadcasted_iota(jnp.int32, (m, _LANES), 1).astype(jnp.float32)
    base = (k * _BK).astype(jnp.float32)
    x2v = x2_ref[...]
    _H = 256

    def _tree(vals, idxs):
        while len(vals) > 1:
            nv, ni = [], []
            for a in range(0, len(vals), 2):
                s = vals[a + 1] < vals[a]
                nv.append(jnp.minimum(vals[a], vals[a + 1]))
                ni.append(jnp.where(s, idxs[a + 1], idxs[a]))
            vals, idxs = nv, ni
        return vals[0], idxs[0]

    def _chunk(h):
        cc = ct_ref[h * _H:(h + 1) * _H, :]
        c2b = jnp.transpose(
            jnp.sum(jnp.square(cc), axis=1, keepdims=True))
        xc = lax.dot_general(
            xm2_ref[...], cc, (((1,), (1,)), ((), ())),
            preferred_element_type=jnp.float32)
        vals = [(x2v + xc[:, t * _LANES:(t + 1) * _LANES])
                + c2b[:, t * _LANES:(t + 1) * _LANES]
                for t in range(_H // _LANES)]
        idxs = [lane + (h * _H + t * _LANES) for t in range(_H // _LANES)]
        return _tree(vals, idxs)

    parts = [_chunk(h) for h in range(_BK // _H)]
    bv, bi = _tree([p[0] for p in parts], [p[1] for p in parts])
    cur_v = minv_ref[...]
    u = bv < cur_v
    minv_ref[...] = jnp.minimum(bv, cur_v)
    mini_ref[...] = jnp.where(u, bi + base, mini_ref[...])

    @pl.when(k == _NSTEPS - 1)
    def _finish():
        mv = minv_ref[...]
        mn = jnp.min(mv, axis=1, keepdims=True)
        cand = jnp.where(mv == mn, mini_ref[...], jnp.float32(_BIG))
        tok = jnp.min(cand, axis=1, keepdims=True)
        tok_ref[...] = tok.astype(jnp.int32)


def _tokens(x, cb):
    m = x.shape[0]
    return pl.pallas_call(
        _argmin_body,
        grid=(_NSTEPS,),
        in_specs=[
            pl.BlockSpec((m, _D), lambda k: (0, 0)),
            pl.BlockSpec((_BK, _D), lambda k: (k, 0)),
        ],
        out_specs=pl.BlockSpec((m, 1), lambda k: (0, 0)),
        out_shape=jax.ShapeDtypeStruct((m, 1), jnp.int32),
        scratch_shapes=[
            pltpu.VMEM((m, _LANES), jnp.float32),
            pltpu.VMEM((m, _LANES), jnp.float32),
            pltpu.VMEM((m, _D), jnp.float32),
            pltpu.VMEM((m, 1), jnp.float32),
        ],
    )(x, cb)


def _sc_gather_body(table_hbm, idx_hbm, out_hbm, idx_v, rows_v, gsem, osem):
    wid = lax.axis_index("s") * _NC + lax.axis_index("c")
    pltpu.sync_copy(idx_hbm.at[wid], idx_v)
    gathers = [
        pltpu.async_copy(
            table_hbm.at[idx_v.at[c]],
            rows_v.at[pl.ds(c * _IDX_CHUNK, _IDX_CHUNK)], gsem)
        for c in range(_NCHUNK)
    ]
    outs = []
    base = _ROWS_PER_W * wid
    for c in range(_NCHUNK):
        gathers[c].wait()
        outs.append(pltpu.async_copy(
            rows_v.at[pl.ds(c * _IDX_CHUNK, _IDX_CHUNK)],
            out_hbm.at[pl.ds(base + c * _IDX_CHUNK, _IDX_CHUNK)], osem))
    for cp in outs:
        cp.wait()


@functools.cache
def _sc_gather():
    return pl.kernel(
        _sc_gather_body,
        out_type=jax.ShapeDtypeStruct((_MG, _D), jnp.float32),
        mesh=plsc.VectorSubcoreMesh(core_axis_name="c", subcore_axis_name="s"),
        scratch_types=[
            pltpu.VMEM((_NCHUNK, _IDX_CHUNK), jnp.int32),
            pltpu.VMEM((_ROWS_PER_W, _D), jnp.float32),
            pltpu.SemaphoreType.DMA,
            pltpu.SemaphoreType.DMA,
        ],
    )


def kernel(inputs, codebook, training):
    shape = inputs.shape
    x = inputs.reshape(_M, _D)
    tok = _tokens(x, codebook)
    idx = tok.reshape(_NW, _NCHUNK, _IDX_CHUNK)
    emb = _sc_gather()(codebook, idx)
    del training
    return emb.reshape(shape)
